# baseline (speedup 1.0000x reference)
import functools

import jax
import jax.numpy as jnp
from jax import lax
from jax.experimental import pallas as pl
from jax.experimental.pallas import tpu as pltpu
from jax.experimental.pallas import tpu_sc as plsc

N_PAT = 1_000_000
N_MEAS = 1000
N_T = 200
L_DIM = 16
N_U = 26
B = 16384

NC, NS = 2, 16
NW = NC * NS
BPW = B // NW


def _gather_body(idx_pat, idx_meas, idx_t, pat_lat, cov_u, meas_lat, time_lat,
                 pat_out, cov_out, meas_out, time_out,
                 idxp_v, idxm_v, idxt_v,
                 sem_p, sem_c, sem_m, sem_t):
    wid = lax.axis_index("s") * NC + lax.axis_index("c")
    base = wid * BPW
    pltpu.sync_copy(idx_pat.at[pl.ds(base, BPW)], idxp_v)
    pltpu.sync_copy(idx_meas.at[pl.ds(base, BPW)], idxm_v)
    pltpu.sync_copy(idx_t.at[pl.ds(base, BPW)], idxt_v)

    def body(g, carry):
        pv = idxp_v[pl.ds(g * 16, 16)]
        mv = idxm_v[pl.ds(g * 16, 16)]
        tv = idxt_v[pl.ds(g * 16, 16)]
        for k in range(16):
            p = pv[k]
            m = mv[k]
            t = tv[k]
            j = base + g * 16 + k
            pltpu.async_copy(pat_lat.at[p], pat_out.at[j], sem_p)
            pltpu.async_copy(cov_u.at[p], cov_out.at[j], sem_c)
            pltpu.async_copy(meas_lat.at[m], meas_out.at[j], sem_m)
            pltpu.async_copy(time_lat.at[t], time_out.at[j], sem_t)
        return carry

    lax.fori_loop(0, BPW // 16, body, 0)
    pltpu.make_async_copy(pat_lat.at[pl.ds(0, BPW)],
                          pat_out.at[pl.ds(base, BPW)], sem_p).wait()
    pltpu.make_async_copy(cov_u.at[pl.ds(0, BPW)],
                          cov_out.at[pl.ds(base, BPW)], sem_c).wait()
    pltpu.make_async_copy(meas_lat.at[pl.ds(0, BPW)],
                          meas_out.at[pl.ds(base, BPW)], sem_m).wait()
    pltpu.make_async_copy(time_lat.at[pl.ds(0, BPW)],
                          time_out.at[pl.ds(base, BPW)], sem_t).wait()


_gather = pl.kernel(
    _gather_body,
    out_type=[
        jax.ShapeDtypeStruct((B, L_DIM), jnp.float32),
        jax.ShapeDtypeStruct((B, N_U), jnp.float32),
        jax.ShapeDtypeStruct((B, L_DIM), jnp.float32),
        jax.ShapeDtypeStruct((B, L_DIM), jnp.float32),
    ],
    mesh=plsc.VectorSubcoreMesh(core_axis_name="c", subcore_axis_name="s"),
    scratch_types=[
        pltpu.VMEM((BPW,), jnp.int32),
        pltpu.VMEM((BPW,), jnp.int32),
        pltpu.VMEM((BPW,), jnp.int32),
        pltpu.SemaphoreType.DMA,
        pltpu.SemaphoreType.DMA,
        pltpu.SemaphoreType.DMA,
        pltpu.SemaphoreType.DMA,
    ],
)


def _tc_body(pat_ref, cov_ref, meas_ref, time_ref, tf_ref, bu_ref, bw_ref,
             out_ref):
    pat = pat_ref[...] + jnp.dot(cov_ref[...], bu_ref[...],
                                 preferred_element_type=jnp.float32)
    tim = time_ref[...] + tf_ref[...] * bw_ref[...]
    out_ref[...] = jnp.sum(pat * meas_ref[...] * tim, axis=1)


def kernel(idx_pat, idx_meas, idx_t, pat_lat, meas_lat, time_lat, beta_u,
           beta_w, covariates_u):
    idx_pat = idx_pat.astype(jnp.int32)
    idx_meas = idx_meas.astype(jnp.int32)
    idx_t = idx_t.astype(jnp.int32)
    pat_r, cov_r, meas_r, time_r = _gather(
        idx_pat, idx_meas, idx_t, pat_lat, covariates_u, meas_lat, time_lat)
    tf = idx_t.astype(jnp.float32).reshape(B, 1)
    pred = pl.pallas_call(
        _tc_body,
        out_shape=jax.ShapeDtypeStruct((B,), jnp.float32),
    )(pat_r, cov_r, meas_r, time_r, tf, beta_u, beta_w)
    return pred


# speedup vs baseline: 2.5846x; 2.5846x over previous
import functools

import jax
import jax.numpy as jnp
from jax import lax
from jax.experimental import pallas as pl
from jax.experimental.pallas import tpu as pltpu
from jax.experimental.pallas import tpu_sc as plsc

N_PAT = 1_000_000
N_MEAS = 1000
N_T = 200
L_DIM = 16
N_U = 26
B = 16384

NC, NS = 2, 16
NW = NC * NS
BPW = B // NW
CH = 128
NCH = BPW // CH


def _gather_body(idx_pat, idx_meas, idx_t, pat_lat, cov_u, meas_lat, time_lat,
                 pat_out, cov_out, meas_out, time_out,
                 idxp_v, idxm_v, idxt_v,
                 pat_v, cov_v, meas_v, time_v,
                 sem_p, sem_c, sem_m, sem_t):
    wid = lax.axis_index("s") * NC + lax.axis_index("c")
    base = wid * BPW
    pltpu.sync_copy(idx_pat.at[pl.ds(base, BPW)], idxp_v)
    pltpu.sync_copy(idx_meas.at[pl.ds(base, BPW)], idxm_v)
    pltpu.sync_copy(idx_t.at[pl.ds(base, BPW)], idxt_v)

    for c in range(NCH):
        def grp(g, carry):
            off = c * CH + g * 16
            pv = idxp_v[pl.ds(off, 16)]
            mv = idxm_v[pl.ds(off, 16)]
            tv = idxt_v[pl.ds(off, 16)]
            for k in range(16):
                p = pv[k]
                m = mv[k]
                t = tv[k]
                i = g * 16 + k
                pltpu.async_copy(pat_lat.at[pl.ds(p, 1)],
                                 pat_v.at[pl.ds(i, 1)], sem_p)
                pltpu.async_copy(cov_u.at[pl.ds(p, 1)],
                                 cov_v.at[pl.ds(i, 1)], sem_c)
                pltpu.async_copy(meas_lat.at[pl.ds(m, 1)],
                                 meas_v.at[pl.ds(i, 1)], sem_m)
                pltpu.async_copy(time_lat.at[pl.ds(t, 1)],
                                 time_v.at[pl.ds(i, 1)], sem_t)
            return carry

        lax.fori_loop(0, CH // 16, grp, 0)
        pltpu.make_async_copy(pat_lat.at[pl.ds(0, CH)], pat_v, sem_p).wait()
        pltpu.make_async_copy(cov_u.at[pl.ds(0, CH)], cov_v, sem_c).wait()
        pltpu.make_async_copy(meas_lat.at[pl.ds(0, CH)], meas_v, sem_m).wait()
        pltpu.make_async_copy(time_lat.at[pl.ds(0, CH)], time_v, sem_t).wait()
        ob = base + c * CH
        pltpu.sync_copy(pat_v, pat_out.at[pl.ds(ob, CH)])
        pltpu.sync_copy(cov_v, cov_out.at[pl.ds(ob, CH)])
        pltpu.sync_copy(meas_v, meas_out.at[pl.ds(ob, CH)])
        pltpu.sync_copy(time_v, time_out.at[pl.ds(ob, CH)])


_gather = pl.kernel(
    _gather_body,
    out_type=[
        jax.ShapeDtypeStruct((B, L_DIM), jnp.float32),
        jax.ShapeDtypeStruct((B, N_U), jnp.float32),
        jax.ShapeDtypeStruct((B, L_DIM), jnp.float32),
        jax.ShapeDtypeStruct((B, L_DIM), jnp.float32),
    ],
    mesh=plsc.VectorSubcoreMesh(core_axis_name="c", subcore_axis_name="s"),
    scratch_types=[
        pltpu.VMEM((BPW,), jnp.int32),
        pltpu.VMEM((BPW,), jnp.int32),
        pltpu.VMEM((BPW,), jnp.int32),
        pltpu.VMEM((CH, L_DIM), jnp.float32),
        pltpu.VMEM((CH, N_U), jnp.float32),
        pltpu.VMEM((CH, L_DIM), jnp.float32),
        pltpu.VMEM((CH, L_DIM), jnp.float32),
        pltpu.SemaphoreType.DMA,
        pltpu.SemaphoreType.DMA,
        pltpu.SemaphoreType.DMA,
        pltpu.SemaphoreType.DMA,
    ],
)


def _tc_body(pat_ref, cov_ref, meas_ref, time_ref, tf_ref, bu_ref, bw_ref,
             out_ref):
    pat = pat_ref[...] + jnp.dot(cov_ref[...], bu_ref[...],
                                 preferred_element_type=jnp.float32)
    tim = time_ref[...] + tf_ref[...] * bw_ref[...]
    out_ref[...] = jnp.sum(pat * meas_ref[...] * tim, axis=1)


def kernel(idx_pat, idx_meas, idx_t, pat_lat, meas_lat, time_lat, beta_u,
           beta_w, covariates_u):
    idx_pat = idx_pat.astype(jnp.int32)
    idx_meas = idx_meas.astype(jnp.int32)
    idx_t = idx_t.astype(jnp.int32)
    pat_r, cov_r, meas_r, time_r = _gather(
        idx_pat, idx_meas, idx_t, pat_lat, covariates_u, meas_lat, time_lat)
    tf = idx_t.astype(jnp.float32).reshape(B, 1)
    pred = pl.pallas_call(
        _tc_body,
        out_shape=jax.ShapeDtypeStruct((B,), jnp.float32),
    )(pat_r, cov_r, meas_r, time_r, tf, beta_u, beta_w)
    return pred
